# baseline (speedup 1.0000x reference)
import functools

import jax
import jax.numpy as jnp
from jax import lax
from jax.experimental import pallas as pl
from jax.experimental.pallas import tpu as pltpu
from jax.experimental.pallas import tpu_sc as plsc

_NC = 8


def _vq_tc_kernel(x_hbm, cb_hbm, idx_ref, x_v, cb_v, dist_s, sems):
    B, D, T = x_v.shape
    K = cb_v.shape[0]
    kc = K // _NC

    def _chunk_copy(c):
        return pltpu.make_async_copy(
            cb_hbm.at[pl.ds(c * kc, kc), :], cb_v.at[pl.ds(c * kc, kc), :],
            sems.at[c])

    x_copy = pltpu.make_async_copy(x_hbm, x_v, sems.at[_NC])
    x_copy.start()
    for c in range(_NC):
        _chunk_copy(c).start()
    x_copy.wait()

    c2_parts = []
    for b in range(B):
        xt = x_v[b].T
        x2 = jnp.sum(xt ** 2, axis=-1, keepdims=True)
        if b == 0:
            for c in range(_NC):
                _chunk_copy(c).wait()
                cbc = cb_v[pl.ds(c * kc, kc), :]
                c2_c = jnp.sum(cbc ** 2, axis=-1)
                c2_parts.append(c2_c)
                mm = jax.lax.dot_general(
                    xt, cbc, (((1,), (1,)), ((), ())),
                    preferred_element_type=jnp.float32)
                dist_s[:, pl.ds(c * kc, kc)] = x2 + c2_c[None, :] - 2.0 * mm
            c2 = jnp.concatenate(c2_parts)
            idx_b = jnp.argmin(dist_s[...], axis=1)
        else:
            cb = cb_v[...]
            mm = jax.lax.dot_general(xt, cb, (((1,), (1,)), ((), ())),
                                     preferred_element_type=jnp.float32)
            dist = x2 + c2[None, :] - 2.0 * mm
            idx_b = jnp.argmin(dist, axis=1)
        idx_ref[0, b] = idx_b


def _make_sc_gather(n_rows, d, n_workers, nc):
    rows_per_w = n_rows // n_workers

    @functools.partial(
        pl.kernel,
        mesh=plsc.VectorSubcoreMesh(core_axis_name="c", subcore_axis_name="s"),
        out_type=jax.ShapeDtypeStruct((n_rows, d), jnp.float32),
        scratch_types=[
            pltpu.VMEM((rows_per_w,), jnp.int32),
            pltpu.VMEM((rows_per_w, d), jnp.float32),
            pltpu.SemaphoreType.DMA,
        ],
    )
    def sc_gather(table_hbm, idx_hbm, out_hbm, idx_v, rows_v, sem):
        wid = lax.axis_index("s") * nc + lax.axis_index("c")
        base = wid * rows_per_w
        pltpu.sync_copy(idx_hbm.at[pl.ds(base, rows_per_w)], idx_v)
        pltpu.async_copy(table_hbm.at[idx_v], rows_v, sem).wait()
        pltpu.sync_copy(rows_v, out_hbm.at[pl.ds(base, rows_per_w)])

    return sc_gather


def kernel(x, codebook):
    B, D, T = x.shape
    K = codebook.shape[0]
    idx3 = pl.pallas_call(
        _vq_tc_kernel,
        in_specs=[pl.BlockSpec(memory_space=pl.ANY),
                  pl.BlockSpec(memory_space=pl.ANY)],
        out_specs=pl.BlockSpec((1, B, T), lambda: (0, 0, 0)),
        out_shape=jax.ShapeDtypeStruct((1, B, T), jnp.int32),
        scratch_shapes=[pltpu.VMEM((B, D, T), jnp.float32),
                        pltpu.VMEM((K, D), jnp.float32),
                        pltpu.VMEM((T, K), jnp.float32),
                        pltpu.SemaphoreType.DMA((_NC + 1,))],
    )(x, codebook)
    idx_flat = idx3.reshape(B * T)
    info = plsc.get_sparse_core_info()
    nw = info.num_cores * info.num_subcores
    rows = _make_sc_gather(B * T, D, nw, info.num_cores)(codebook, idx_flat)
    quantized = jnp.transpose(rows.reshape(B, T, D), (0, 2, 1))
    return quantized, idx_flat.reshape(B, T)


# speedup vs baseline: 1.0256x; 1.0256x over previous
import functools

import jax
import jax.numpy as jnp
from jax import lax
from jax.experimental import pallas as pl
from jax.experimental.pallas import tpu as pltpu
from jax.experimental.pallas import tpu_sc as plsc


def _vq_tc_kernel(x_ref, cb_ref, idx_ref):
    xt = x_ref[0].T
    cb = cb_ref[...]
    mm = jax.lax.dot_general(xt, cb, (((1,), (1,)), ((), ())),
                             preferred_element_type=jnp.float32)
    x2 = jnp.sum(xt ** 2, axis=-1, keepdims=True)
    c2 = jnp.sum(cb ** 2, axis=-1)
    dist = x2 + c2[None, :] - 2.0 * mm
    idx_ref[0, 0] = jnp.argmin(dist, axis=1)


def _make_sc_gather(n_rows, d, n_workers, nc):
    rows_per_w = n_rows // n_workers

    @functools.partial(
        pl.kernel,
        mesh=plsc.VectorSubcoreMesh(core_axis_name="c", subcore_axis_name="s"),
        out_type=jax.ShapeDtypeStruct((n_rows, d), jnp.float32),
        scratch_types=[
            pltpu.VMEM((rows_per_w,), jnp.int32),
            pltpu.VMEM((rows_per_w, d), jnp.float32),
            pltpu.SemaphoreType.DMA,
        ],
    )
    def sc_gather(table_hbm, idx_hbm, out_hbm, idx_v, rows_v, sem):
        wid = lax.axis_index("s") * nc + lax.axis_index("c")
        base = wid * rows_per_w
        pltpu.sync_copy(idx_hbm.at[pl.ds(base, rows_per_w)], idx_v)
        pltpu.async_copy(table_hbm.at[idx_v], rows_v, sem).wait()
        pltpu.sync_copy(rows_v, out_hbm.at[pl.ds(base, rows_per_w)])

    return sc_gather


def kernel(x, codebook):
    B, D, T = x.shape
    K = codebook.shape[0]
    idx3 = pl.pallas_call(
        _vq_tc_kernel,
        grid=(B,),
        in_specs=[pl.BlockSpec((1, D, T), lambda b: (b, 0, 0)),
                  pl.BlockSpec(memory_space=pltpu.MemorySpace.VMEM)],
        out_specs=pl.BlockSpec((1, 1, T), lambda b: (b, 0, 0)),
        out_shape=jax.ShapeDtypeStruct((B, 1, T), jnp.int32),
    )(x, codebook)
    idx_flat = idx3.reshape(B * T)
    info = plsc.get_sparse_core_info()
    nw = info.num_cores * info.num_subcores
    rows = _make_sc_gather(B * T, D, nw, info.num_cores)(codebook, idx_flat)
    quantized = jnp.transpose(rows.reshape(B, T, D), (0, 2, 1))
    return quantized, idx_flat.reshape(B, T)


# speedup vs baseline: 1.0436x; 1.0175x over previous
import functools

import jax
import jax.numpy as jnp
from jax import lax
from jax.experimental import pallas as pl
from jax.experimental.pallas import tpu as pltpu
from jax.experimental.pallas import tpu_sc as plsc


def _vq_tc_kernel(x_ref, cb_ref, idx_ref, c2_s):
    b = pl.program_id(0)
    xt = x_ref[0].T
    cb = cb_ref[...]

    @pl.when(b == 0)
    def _():
        c2_s[...] = jnp.sum(cb ** 2, axis=-1, keepdims=True).T

    mm = jax.lax.dot_general(xt, cb, (((1,), (1,)), ((), ())),
                             preferred_element_type=jnp.float32)
    x2 = jnp.sum(xt ** 2, axis=-1, keepdims=True)
    dist = x2 + c2_s[...] - 2.0 * mm
    idx_ref[0, 0] = jnp.argmin(dist, axis=1)


def _make_sc_gather(n_rows, d, n_workers, nc):
    rows_per_w = n_rows // n_workers

    @functools.partial(
        pl.kernel,
        mesh=plsc.VectorSubcoreMesh(core_axis_name="c", subcore_axis_name="s"),
        out_type=jax.ShapeDtypeStruct((n_rows, d), jnp.float32),
        scratch_types=[
            pltpu.VMEM((rows_per_w,), jnp.int32),
            pltpu.VMEM((rows_per_w, d), jnp.float32),
            pltpu.SemaphoreType.DMA,
        ],
    )
    def sc_gather(table_hbm, idx_hbm, out_hbm, idx_v, rows_v, sem):
        wid = lax.axis_index("s") * nc + lax.axis_index("c")
        base = wid * rows_per_w
        pltpu.sync_copy(idx_hbm.at[pl.ds(base, rows_per_w)], idx_v)
        pltpu.async_copy(table_hbm.at[idx_v], rows_v, sem).wait()
        pltpu.sync_copy(rows_v, out_hbm.at[pl.ds(base, rows_per_w)])

    return sc_gather


def kernel(x, codebook):
    B, D, T = x.shape
    K = codebook.shape[0]
    idx3 = pl.pallas_call(
        _vq_tc_kernel,
        grid=(B,),
        in_specs=[pl.BlockSpec((1, D, T), lambda b: (b, 0, 0)),
                  pl.BlockSpec((K, D), lambda b: (0, 0))],
        out_specs=pl.BlockSpec((1, 1, T), lambda b: (b, 0, 0)),
        out_shape=jax.ShapeDtypeStruct((B, 1, T), jnp.int32),
        scratch_shapes=[pltpu.VMEM((1, K), jnp.float32)],
    )(x, codebook)
    idx_flat = idx3.reshape(B * T)
    info = plsc.get_sparse_core_info()
    nw = info.num_cores * info.num_subcores
    rows = _make_sc_gather(B * T, D, nw, info.num_cores)(codebook, idx_flat)
    quantized = jnp.transpose(rows.reshape(B, T, D), (0, 2, 1))
    return quantized, idx_flat.reshape(B, T)
